# baseline (speedup 1.0000x reference)
import functools

import numpy as np
import jax
import jax.numpy as jnp
from jax.experimental import pallas as pl
from jax.experimental.pallas import tpu as pltpu
from jax.experimental.pallas import tpu_sc as plsc

_K = 5
_MARGIN = 0.2
_CUTOFF = 0.5
_NZCUT = 1.4


def _row_block(n, cap):
    best = 8
    for b in range(8, cap + 1, 8):
        if n % b == 0:
            best = b
    return best


def _logw_block(xi, xa, row0):
    BR, d = xi.shape
    n = xa.shape[0]
    G = jax.lax.dot_general(xi, xa, (((1,), (1,)), ((), ())),
                            preferred_element_type=jnp.float32)
    sqi = jnp.sum(xi * xi, axis=1, keepdims=True)
    ones = jnp.ones((8, d), jnp.float32)
    sqa = jax.lax.dot_general(ones, xa * xa, (((1,), (1,)), ((), ())),
                              preferred_element_type=jnp.float32)[0:1]
    rows = row0 + jax.lax.broadcasted_iota(jnp.int32, (BR, n), 0)
    cols = jax.lax.broadcasted_iota(jnp.int32, (BR, n), 1)
    dist2 = sqi + sqa - 2.0 * G + jnp.where(rows == cols, 1.0, 0.0)
    dis = jnp.sqrt(jnp.maximum(dist2, 1e-12))
    dis = jnp.maximum(dis, _CUTOFF)
    log_w = ((2.0 - float(d)) * jnp.log(dis)
             - (float(d - 3) / 2.0) * jnp.log(jnp.maximum(1.0 - 0.25 * dis * dis, 1e-8)))
    keep = jnp.logical_and(rows // _K != cols // _K, dis < _NZCUT)
    return log_w, keep


def _stats_body(xi_ref, xa_ref, m_ref, s_ref, raw_ref):
    i = pl.program_id(0)
    xi = xi_ref[...]
    xa = xa_ref[...]
    BR = xi.shape[0]
    log_w, keep = _logw_block(xi, xa, i * BR)
    raw = jnp.max(log_w, axis=1, keepdims=True)
    mker = jnp.where(keep, log_w, -1e30)
    m = jnp.max(mker, axis=1, keepdims=True)
    e = jnp.where(keep, jnp.exp(log_w - m), 0.0)
    s = jnp.sum(e, axis=1, keepdims=True)
    m_ref[...] = jnp.broadcast_to(m, m_ref.shape)
    s_ref[...] = jnp.broadcast_to(s, s_ref.shape)
    raw_ref[...] = jnp.broadcast_to(raw, raw_ref.shape)


def _gumbel_block(m):
    ks1 = jnp.int32(42)
    ks2 = jnp.int32(42 ^ 0x1BD11BDA)

    def rot(x, c):
        return jax.lax.shift_left(x, jnp.int32(c)) | jax.lax.shift_right_logical(
            x, jnp.int32(32 - c))

    def rounds(x0, x1, rots):
        for c in rots:
            x0 = x0 + x1
            x1 = rot(x1, c)
            x1 = x0 ^ x1
        return x0, x1

    x1 = m + ks1
    x0 = x1
    x1 = rot(x1, 13) ^ x0
    x0, x1 = rounds(x0, x1, (15, 26, 6))
    x0, x1 = x0 + ks1, x1 + ks2 + jnp.int32(1)
    x0, x1 = rounds(x0, x1, (17, 29, 16, 24))
    x0, x1 = x0 + ks2, x1 + jnp.int32(2)
    x0, x1 = rounds(x0, x1, (13, 15, 26, 6))
    x0, x1 = x0, x1 + ks1 + jnp.int32(3)
    x0, x1 = rounds(x0, x1, (17, 29, 16, 24))
    x0, x1 = x0 + ks1, x1 + ks2 + jnp.int32(4)
    x0, x1 = rounds(x0, x1, (13, 15, 26, 6))
    x0, x1 = x0 + ks2, x1 + jnp.int32(5)
    bits = x0 ^ x1

    fb = jax.lax.shift_right_logical(bits, jnp.int32(9)) | jnp.int32(0x3F800000)
    floats = jax.lax.bitcast_convert_type(fb, jnp.float32) - jnp.float32(1.0)
    tiny = jnp.float32(np.finfo(np.float32).tiny)
    u = jax.lax.max(tiny, floats + tiny)
    return -jnp.log(-jnp.log(u))


def _sample_body(c_ref, mm_ref, xi_ref, xa_ref, idx_ref, lin_s, cols_s):
    i = pl.program_id(0)
    xi = xi_ref[...]
    BR = xi.shape[0]
    n = xa_ref.shape[0]

    @pl.when(i == 0)
    def _():
        rl = jax.lax.broadcasted_iota(jnp.int32, (BR, n), 0)
        cl = jax.lax.broadcasted_iota(jnp.int32, (BR, n), 1)
        lin_s[...] = rl * n + cl
        cols_s[...] = cl

    log_w, keep = _logw_block(xi, xa_ref[...], i * BR)
    w = jnp.where(keep, jnp.exp(log_w - mm_ref[0, 0]), 0.0)
    wn = w / c_ref[...]
    logits = jnp.log(wn + 1e-12)

    lin = lin_s[...]
    cols = cols_s[...]
    out = []
    for r in range(_K - 1):
        flat = lin + (r * n + i * BR) * n
        vals = logits + _gumbel_block(flat)
        mx = jnp.max(vals, axis=1, keepdims=True)
        idx = jnp.min(jnp.where(vals == mx, cols, n), axis=1, keepdims=True)
        out.append(idx[None])
    idx_ref[...] = jnp.concatenate(out, axis=0)


def _sample_negative_indices(xs):
    n, d = xs.shape
    BR1 = _row_block(n, 256)
    m, s, raw = pl.pallas_call(
        _stats_body,
        grid=(n // BR1,),
        in_specs=[
            pl.BlockSpec((BR1, d), lambda i: (i, 0)),
            pl.BlockSpec((n, d), lambda i: (0, 0)),
        ],
        out_specs=[
            pl.BlockSpec((BR1, 128), lambda i: (i, 0)),
            pl.BlockSpec((BR1, 128), lambda i: (i, 0)),
            pl.BlockSpec((BR1, 128), lambda i: (i, 0)),
        ],
        out_shape=[jax.ShapeDtypeStruct((n, 128), jnp.float32)] * 3,
    )(xs, xs)
    mm = jnp.max(raw[:, 0]).reshape(1, 1)
    c = s[:, :1] * jnp.exp(m[:, :1] - mm) + 1e-8

    BR2 = _row_block(n, 128)
    idx = pl.pallas_call(
        _sample_body,
        grid=(n // BR2,),
        in_specs=[
            pl.BlockSpec((BR2, 1), lambda i: (i, 0)),
            pl.BlockSpec((1, 1), lambda i: (0, 0)),
            pl.BlockSpec((BR2, d), lambda i: (i, 0)),
            pl.BlockSpec((n, d), lambda i: (0, 0)),
        ],
        out_specs=pl.BlockSpec((_K - 1, BR2, 1), lambda i: (0, i, 0)),
        out_shape=jax.ShapeDtypeStruct((_K - 1, n, 1), jnp.int32),
        scratch_shapes=[pltpu.VMEM((BR2, n), jnp.int32),
                        pltpu.VMEM((BR2, n), jnp.int32)],
    )(c, mm, xs, xs)
    return idx[:, :, 0].T.reshape(-1)


def _sc_sqrt(z):
    bz = plsc.bitcast(z, jnp.int32)
    y = plsc.bitcast(jnp.int32(0x5F3759DF) - jax.lax.shift_right_logical(bz, jnp.int32(1)),
                     jnp.float32)
    half = jnp.float32(0.5) * z
    for _ in range(3):
        y = y * (jnp.float32(1.5) - half * y * y)
    return z * y


def _make_sc_loss(n, d, margin):
    info = plsc.get_sparse_core_info()
    nw = info.num_cores * info.num_subcores
    apw = n // nw
    tpw = apw * (_K - 1)
    chunks = -(-tpw // 128)
    tpad = chunks * 128
    iters = tpad // 16
    slab_rows = min(n, ((apw + 7) // 8) * 8 + 8)
    mesh = plsc.VectorSubcoreMesh(core_axis_name="c", subcore_axis_name="s")

    @functools.partial(
        pl.kernel,
        mesh=mesh,
        compiler_params=pltpu.CompilerParams(needs_layout_passes=False),
        out_type=[
            jax.ShapeDtypeStruct((nw * 16,), jnp.float32),
            jax.ShapeDtypeStruct((nw * 16,), jnp.int32),
        ],
        scratch_types=[
            pltpu.VMEM((slab_rows, d), jnp.float32),
            pltpu.VMEM((tpad, d), jnp.float32),
            pltpu.VMEM((chunks, 128), jnp.int32),
            pltpu.VMEM((n,), jnp.float32),
            pltpu.VMEM((16,), jnp.float32),
            pltpu.VMEM((16,), jnp.int32),
            pltpu.SemaphoreType.DMA,
        ],
    )
    def sc_loss(x_hbm, nidx_hbm, beta_hbm, sums_hbm, cnts_hbm,
                slab_v, negs_v, idx_v, beta_v, sout_v, cout_v, sem):
        wid = jax.lax.axis_index("s") * info.num_cores + jax.lax.axis_index("c")
        base_a = wid * apw
        slab0 = pl.multiple_of(
            jnp.minimum((base_a // 8) * 8, n - slab_rows), 8)
        off = base_a - slab0
        pltpu.sync_copy(x_hbm.at[pl.ds(slab0, slab_rows)], slab_v)
        pltpu.sync_copy(nidx_hbm.at[wid], idx_v)
        pltpu.sync_copy(beta_hbm, beta_v)
        for cchunk in range(chunks):
            pltpu.async_copy(x_hbm.at[idx_v.at[cchunk]],
                             negs_v.at[pl.ds(cchunk * 128, 128)], sem).wait()

        lane = jax.lax.iota(jnp.int32, 16)

        def tri_body(it, carry):
            s_acc, c_acc = carry
            t16 = it * 16 + lane
            a_loc = jax.lax.shift_right_logical(t16, jnp.int32(2))
            rr = jax.lax.bitwise_and(t16, jnp.int32(3))
            blk = jax.lax.shift_right_logical(a_loc * jnp.int32(52429), jnp.int32(18))
            pos_in = a_loc - blk * jnp.int32(_K)
            mth = jnp.where(rr >= pos_in, rr + jnp.int32(1), rr)
            p_loc = blk * jnp.int32(_K) + mth
            beta16 = plsc.load_gather(beta_v, [base_a + a_loc])

            a_row = off + a_loc
            p_row = off + p_loc

            def dim_body(dc, dcarry):
                ap, an = dcarry
                for u in range(8):
                    dsplat = jnp.full((16,), dc * 8 + u, jnp.int32)
                    a_d = plsc.load_gather(slab_v, [a_row, dsplat])
                    p_d = plsc.load_gather(slab_v, [p_row, dsplat])
                    n_d = plsc.load_gather(negs_v, [t16, dsplat])
                    ap = ap + (a_d - p_d) * (a_d - p_d)
                    an = an + (a_d - n_d) * (a_d - n_d)
                return (ap, an)

            zero = jnp.zeros((16,), jnp.float32)
            ap, an = plsc.parallel_loop(0, d // 8, carry=(zero, zero))(dim_body)
            d_ap = _sc_sqrt(ap + jnp.float32(1e-8))
            d_an = _sc_sqrt(an + jnp.float32(1e-8))
            pos = jnp.maximum(d_ap - beta16 + jnp.float32(margin), jnp.float32(0.0))
            neg = jnp.maximum(beta16 - d_an + jnp.float32(margin), jnp.float32(0.0))
            valid = t16 < jnp.int32(tpw)
            hit = jnp.logical_and(valid,
                                  jnp.logical_or(pos > jnp.float32(0.0),
                                                 neg > jnp.float32(0.0)))
            s_acc = s_acc + jnp.where(valid, pos + neg, jnp.float32(0.0))
            c_acc = c_acc + jnp.where(hit, jnp.int32(1), jnp.int32(0))
            return (s_acc, c_acc)

        zf = jnp.zeros((16,), jnp.float32)
        zi = jnp.zeros((16,), jnp.int32)
        s_acc, c_acc = jax.lax.fori_loop(0, iters, tri_body, (zf, zi))
        sout_v[...] = s_acc
        cout_v[...] = c_acc
        pltpu.sync_copy(sout_v, sums_hbm.at[pl.ds(wid * 16, 16)])
        pltpu.sync_copy(cout_v, cnts_hbm.at[pl.ds(wid * 16, 16)])

    return sc_loss


def _triplet_indices(n, k):
    a_idx = np.repeat(np.arange(n), k - 1)
    blocks = np.arange(n) // k
    offs = np.arange(k)
    p_full = blocks[:, None] * k + offs[None, :]
    p_keep = p_full != np.arange(n)[:, None]
    p_idx = p_full[p_keep]
    return a_idx, p_idx


def kernel(x, y, beta_in):
    n, d = x.shape
    xs = jax.lax.stop_gradient(x)
    n_index = _sample_negative_indices(xs)

    info = plsc.get_sparse_core_info()
    nw = info.num_cores * info.num_subcores
    tpw = (n // nw) * (_K - 1)
    chunks = -(-tpw // 128)
    nidx = jnp.pad(n_index.reshape(nw, tpw), ((0, 0), (0, chunks * 128 - tpw))
                   ).reshape(nw, chunks, 128).astype(jnp.int32)
    sums, cnts = _make_sc_loss(n, d, _MARGIN)(x, nidx, beta_in)
    pair_cnt = jnp.sum(cnts)
    return jnp.sum(sums) / pair_cnt.astype(jnp.float32)


# speedup vs baseline: 1.0540x; 1.0540x over previous
import functools

import numpy as np
import jax
import jax.numpy as jnp
from jax.experimental import pallas as pl
from jax.experimental.pallas import tpu as pltpu
from jax.experimental.pallas import tpu_sc as plsc

_K = 5
_MARGIN = 0.2
_CUTOFF = 0.5
_NZCUT = 1.4


def _row_block(n, cap):
    best = 8
    for b in range(8, cap + 1, 8):
        if n % b == 0:
            best = b
    return best


def _logw_block(xi, xa, row0):
    BR, d = xi.shape
    n = xa.shape[0]
    G = jax.lax.dot_general(xi, xa, (((1,), (1,)), ((), ())),
                            preferred_element_type=jnp.float32)
    sqi = jnp.sum(xi * xi, axis=1, keepdims=True)
    ones = jnp.ones((8, d), jnp.float32)
    sqa = jax.lax.dot_general(ones, xa * xa, (((1,), (1,)), ((), ())),
                              preferred_element_type=jnp.float32)[0:1]
    rows = row0 + jax.lax.broadcasted_iota(jnp.int32, (BR, n), 0)
    cols = jax.lax.broadcasted_iota(jnp.int32, (BR, n), 1)
    dist2 = sqi + sqa - 2.0 * G + jnp.where(rows == cols, 1.0, 0.0)
    dis = jnp.sqrt(jnp.maximum(dist2, 1e-12))
    dis = jnp.maximum(dis, _CUTOFF)
    log_w = ((2.0 - float(d)) * jnp.log(dis)
             - (float(d - 3) / 2.0) * jnp.log(jnp.maximum(1.0 - 0.25 * dis * dis, 1e-8)))
    keep = jnp.logical_and(rows // _K != cols // _K, dis < _NZCUT)
    return log_w, keep


def _stats_body(xi_ref, xa_ref, m_ref, s_ref, raw_ref):
    i = pl.program_id(0)
    xi = xi_ref[...]
    xa = xa_ref[...]
    BR = xi.shape[0]
    log_w, keep = _logw_block(xi, xa, i * BR)
    raw = jnp.max(log_w, axis=1, keepdims=True)
    mker = jnp.where(keep, log_w, -1e30)
    m = jnp.max(mker, axis=1, keepdims=True)
    e = jnp.where(keep, jnp.exp(log_w - m), 0.0)
    s = jnp.sum(e, axis=1, keepdims=True)
    m_ref[...] = jnp.broadcast_to(m, m_ref.shape)
    s_ref[...] = jnp.broadcast_to(s, s_ref.shape)
    raw_ref[...] = jnp.broadcast_to(raw, raw_ref.shape)


def _gumbel_block(m):
    ks1 = jnp.int32(42)
    ks2 = jnp.int32(42 ^ 0x1BD11BDA)

    def rot(x, c):
        return jax.lax.shift_left(x, jnp.int32(c)) | jax.lax.shift_right_logical(
            x, jnp.int32(32 - c))

    def rounds(x0, x1, rots):
        for c in rots:
            x0 = x0 + x1
            x1 = rot(x1, c)
            x1 = x0 ^ x1
        return x0, x1

    x1 = m + ks1
    x0 = x1
    x1 = rot(x1, 13) ^ x0
    x0, x1 = rounds(x0, x1, (15, 26, 6))
    x0, x1 = x0 + ks1, x1 + ks2 + jnp.int32(1)
    x0, x1 = rounds(x0, x1, (17, 29, 16, 24))
    x0, x1 = x0 + ks2, x1 + jnp.int32(2)
    x0, x1 = rounds(x0, x1, (13, 15, 26, 6))
    x0, x1 = x0, x1 + ks1 + jnp.int32(3)
    x0, x1 = rounds(x0, x1, (17, 29, 16, 24))
    x0, x1 = x0 + ks1, x1 + ks2 + jnp.int32(4)
    x0, x1 = rounds(x0, x1, (13, 15, 26, 6))
    x0, x1 = x0 + ks2, x1 + jnp.int32(5)
    bits = x0 ^ x1

    fb = jax.lax.shift_right_logical(bits, jnp.int32(9)) | jnp.int32(0x3F800000)
    floats = jax.lax.bitcast_convert_type(fb, jnp.float32) - jnp.float32(1.0)
    tiny = jnp.float32(np.finfo(np.float32).tiny)
    u = jax.lax.max(tiny, floats + tiny)
    return -jnp.log(-jnp.log(u))


def _sample_body(c_ref, mm_ref, xi_ref, xa_ref, g_ref, idx_ref, logits_s):
    i = pl.program_id(0)
    xi = xi_ref[...]
    BR = xi.shape[0]
    n = xa_ref.shape[0]

    @pl.when(pl.program_id(1) == 0)
    def _():
        log_w, keep = _logw_block(xi, xa_ref[...], i * BR)
        w = jnp.where(keep, jnp.exp(log_w - mm_ref[0, 0]), 0.0)
        wn = w / c_ref[...]
        logits_s[...] = jnp.log(wn + 1e-12)

    vals = logits_s[...] + g_ref[0]
    mx = jnp.max(vals, axis=1, keepdims=True)
    cols = jax.lax.broadcasted_iota(jnp.int32, vals.shape, 1)
    idx = jnp.min(jnp.where(vals == mx, cols, n), axis=1, keepdims=True)
    idx_ref[0] = idx


def _sample_negative_indices(xs):
    n, d = xs.shape
    BR1 = _row_block(n, 256)
    m, s, raw = pl.pallas_call(
        _stats_body,
        grid=(n // BR1,),
        in_specs=[
            pl.BlockSpec((BR1, d), lambda i: (i, 0)),
            pl.BlockSpec((n, d), lambda i: (0, 0)),
        ],
        out_specs=[
            pl.BlockSpec((BR1, 128), lambda i: (i, 0)),
            pl.BlockSpec((BR1, 128), lambda i: (i, 0)),
            pl.BlockSpec((BR1, 128), lambda i: (i, 0)),
        ],
        out_shape=[jax.ShapeDtypeStruct((n, 128), jnp.float32)] * 3,
    )(xs, xs)
    mm = jnp.max(raw[:, 0]).reshape(1, 1)
    c = s[:, :1] * jnp.exp(m[:, :1] - mm) + 1e-8

    g = jax.random.gumbel(jax.random.key(42), (_K - 1, n, n), jnp.float32)

    BR2 = _row_block(n, 128)
    idx = pl.pallas_call(
        _sample_body,
        grid=(n // BR2, _K - 1),
        in_specs=[
            pl.BlockSpec((BR2, 1), lambda i, r: (i, 0)),
            pl.BlockSpec((1, 1), lambda i, r: (0, 0)),
            pl.BlockSpec((BR2, d), lambda i, r: (i, 0)),
            pl.BlockSpec((n, d), lambda i, r: (0, 0)),
            pl.BlockSpec((1, BR2, n), lambda i, r: (r, i, 0)),
        ],
        out_specs=pl.BlockSpec((1, BR2, 1), lambda i, r: (r, i, 0)),
        out_shape=jax.ShapeDtypeStruct((_K - 1, n, 1), jnp.int32),
        scratch_shapes=[pltpu.VMEM((BR2, n), jnp.float32)],
    )(c, mm, xs, xs, g)
    return idx[:, :, 0].T.reshape(-1)


def _sc_sqrt(z):
    bz = plsc.bitcast(z, jnp.int32)
    y = plsc.bitcast(jnp.int32(0x5F3759DF) - jax.lax.shift_right_logical(bz, jnp.int32(1)),
                     jnp.float32)
    half = jnp.float32(0.5) * z
    for _ in range(3):
        y = y * (jnp.float32(1.5) - half * y * y)
    return z * y


def _make_sc_loss(n, d, margin):
    info = plsc.get_sparse_core_info()
    nw = info.num_cores * info.num_subcores
    apw = n // nw
    tpw = apw * (_K - 1)
    chunks = -(-tpw // 128)
    tpad = chunks * 128
    iters = tpad // 16
    slab_rows = min(n, ((apw + 7) // 8) * 8 + 8)
    mesh = plsc.VectorSubcoreMesh(core_axis_name="c", subcore_axis_name="s")

    @functools.partial(
        pl.kernel,
        mesh=mesh,
        compiler_params=pltpu.CompilerParams(needs_layout_passes=False),
        out_type=[
            jax.ShapeDtypeStruct((nw * 16,), jnp.float32),
            jax.ShapeDtypeStruct((nw * 16,), jnp.int32),
        ],
        scratch_types=[
            pltpu.VMEM((slab_rows, d), jnp.float32),
            pltpu.VMEM((tpad, d), jnp.float32),
            pltpu.VMEM((chunks, 128), jnp.int32),
            pltpu.VMEM((n,), jnp.float32),
            pltpu.VMEM((16,), jnp.float32),
            pltpu.VMEM((16,), jnp.int32),
            pltpu.SemaphoreType.DMA,
        ],
    )
    def sc_loss(x_hbm, nidx_hbm, beta_hbm, sums_hbm, cnts_hbm,
                slab_v, negs_v, idx_v, beta_v, sout_v, cout_v, sem):
        wid = jax.lax.axis_index("s") * info.num_cores + jax.lax.axis_index("c")
        base_a = wid * apw
        slab0 = pl.multiple_of(
            jnp.minimum((base_a // 8) * 8, n - slab_rows), 8)
        off = base_a - slab0
        pltpu.sync_copy(x_hbm.at[pl.ds(slab0, slab_rows)], slab_v)
        pltpu.sync_copy(nidx_hbm.at[wid], idx_v)
        pltpu.sync_copy(beta_hbm, beta_v)
        for cchunk in range(chunks):
            pltpu.async_copy(x_hbm.at[idx_v.at[cchunk]],
                             negs_v.at[pl.ds(cchunk * 128, 128)], sem).wait()

        lane = jax.lax.iota(jnp.int32, 16)

        def tri_body(it, carry):
            s_acc, c_acc = carry
            t16 = it * 16 + lane
            a_loc = jax.lax.shift_right_logical(t16, jnp.int32(2))
            rr = jax.lax.bitwise_and(t16, jnp.int32(3))
            blk = jax.lax.shift_right_logical(a_loc * jnp.int32(52429), jnp.int32(18))
            pos_in = a_loc - blk * jnp.int32(_K)
            mth = jnp.where(rr >= pos_in, rr + jnp.int32(1), rr)
            p_loc = blk * jnp.int32(_K) + mth
            beta16 = plsc.load_gather(beta_v, [base_a + a_loc])

            a_row = off + a_loc
            p_row = off + p_loc

            def dim_body(dc, dcarry):
                ap, an = dcarry
                for u in range(8):
                    dsplat = jnp.full((16,), dc * 8 + u, jnp.int32)
                    a_d = plsc.load_gather(slab_v, [a_row, dsplat])
                    p_d = plsc.load_gather(slab_v, [p_row, dsplat])
                    n_d = plsc.load_gather(negs_v, [t16, dsplat])
                    ap = ap + (a_d - p_d) * (a_d - p_d)
                    an = an + (a_d - n_d) * (a_d - n_d)
                return (ap, an)

            zero = jnp.zeros((16,), jnp.float32)
            ap, an = plsc.parallel_loop(0, d // 8, carry=(zero, zero))(dim_body)
            d_ap = _sc_sqrt(ap + jnp.float32(1e-8))
            d_an = _sc_sqrt(an + jnp.float32(1e-8))
            pos = jnp.maximum(d_ap - beta16 + jnp.float32(margin), jnp.float32(0.0))
            neg = jnp.maximum(beta16 - d_an + jnp.float32(margin), jnp.float32(0.0))
            valid = t16 < jnp.int32(tpw)
            hit = jnp.logical_and(valid,
                                  jnp.logical_or(pos > jnp.float32(0.0),
                                                 neg > jnp.float32(0.0)))
            s_acc = s_acc + jnp.where(valid, pos + neg, jnp.float32(0.0))
            c_acc = c_acc + jnp.where(hit, jnp.int32(1), jnp.int32(0))
            return (s_acc, c_acc)

        zf = jnp.zeros((16,), jnp.float32)
        zi = jnp.zeros((16,), jnp.int32)
        s_acc, c_acc = jax.lax.fori_loop(0, iters, tri_body, (zf, zi))
        sout_v[...] = s_acc
        cout_v[...] = c_acc
        pltpu.sync_copy(sout_v, sums_hbm.at[pl.ds(wid * 16, 16)])
        pltpu.sync_copy(cout_v, cnts_hbm.at[pl.ds(wid * 16, 16)])

    return sc_loss


def _triplet_indices(n, k):
    a_idx = np.repeat(np.arange(n), k - 1)
    blocks = np.arange(n) // k
    offs = np.arange(k)
    p_full = blocks[:, None] * k + offs[None, :]
    p_keep = p_full != np.arange(n)[:, None]
    p_idx = p_full[p_keep]
    return a_idx, p_idx


def kernel(x, y, beta_in):
    n, d = x.shape
    xs = jax.lax.stop_gradient(x)
    n_index = _sample_negative_indices(xs)

    info = plsc.get_sparse_core_info()
    nw = info.num_cores * info.num_subcores
    tpw = (n // nw) * (_K - 1)
    chunks = -(-tpw // 128)
    nidx = jnp.pad(n_index.reshape(nw, tpw), ((0, 0), (0, chunks * 128 - tpw))
                   ).reshape(nw, chunks, 128).astype(jnp.int32)
    sums, cnts = _make_sc_loss(n, d, _MARGIN)(x, nidx, beta_in)
    pair_cnt = jnp.sum(cnts)
    return jnp.sum(sums) / pair_cnt.astype(jnp.float32)


# speedup vs baseline: 1.0562x; 1.0022x over previous
import functools

import numpy as np
import jax
import jax.numpy as jnp
from jax.experimental import pallas as pl
from jax.experimental.pallas import tpu as pltpu
from jax.experimental.pallas import tpu_sc as plsc

_K = 5
_MARGIN = 0.2
_CUTOFF = 0.5
_NZCUT = 1.4


def _row_block(n, cap):
    best = 8
    for b in range(8, cap + 1, 8):
        if n % b == 0:
            best = b
    return best


def _logw_block(xi, xa, row0):
    BR, d = xi.shape
    n = xa.shape[0]
    G = jax.lax.dot_general(xi, xa, (((1,), (1,)), ((), ())),
                            preferred_element_type=jnp.float32)
    sqi = jnp.sum(xi * xi, axis=1, keepdims=True)
    ones = jnp.ones((8, d), jnp.float32)
    sqa = jax.lax.dot_general(ones, xa * xa, (((1,), (1,)), ((), ())),
                              preferred_element_type=jnp.float32)[0:1]
    rows = row0 + jax.lax.broadcasted_iota(jnp.int32, (BR, n), 0)
    cols = jax.lax.broadcasted_iota(jnp.int32, (BR, n), 1)
    dist2 = sqi + sqa - 2.0 * G + jnp.where(rows == cols, 1.0, 0.0)
    dis = jnp.sqrt(jnp.maximum(dist2, 1e-12))
    dis = jnp.maximum(dis, _CUTOFF)
    log_w = ((2.0 - float(d)) * jnp.log(dis)
             - (float(d - 3) / 2.0) * jnp.log(jnp.maximum(1.0 - 0.25 * dis * dis, 1e-8)))
    keep = jnp.logical_and(rows // _K != cols // _K, dis < _NZCUT)
    return log_w, keep


def _stats_body(xi_ref, xa_ref, m_ref, s_ref, raw_ref):
    i = pl.program_id(0)
    xi = xi_ref[...]
    xa = xa_ref[...]
    BR = xi.shape[0]
    log_w, keep = _logw_block(xi, xa, i * BR)
    raw = jnp.max(log_w, axis=1, keepdims=True)
    mker = jnp.where(keep, log_w, -1e30)
    m = jnp.max(mker, axis=1, keepdims=True)
    e = jnp.where(keep, jnp.exp(log_w - m), 0.0)
    s = jnp.sum(e, axis=1, keepdims=True)
    m_ref[...] = jnp.broadcast_to(m, m_ref.shape)
    s_ref[...] = jnp.broadcast_to(s, s_ref.shape)
    raw_ref[...] = jnp.broadcast_to(raw, raw_ref.shape)


def _gumbel_block(m):
    ks1 = jnp.int32(42)
    ks2 = jnp.int32(42 ^ 0x1BD11BDA)

    def rot(x, c):
        return jax.lax.shift_left(x, jnp.int32(c)) | jax.lax.shift_right_logical(
            x, jnp.int32(32 - c))

    def rounds(x0, x1, rots):
        for c in rots:
            x0 = x0 + x1
            x1 = rot(x1, c)
            x1 = x0 ^ x1
        return x0, x1

    x1 = m + ks1
    x0 = x1
    x1 = rot(x1, 13) ^ x0
    x0, x1 = rounds(x0, x1, (15, 26, 6))
    x0, x1 = x0 + ks1, x1 + ks2 + jnp.int32(1)
    x0, x1 = rounds(x0, x1, (17, 29, 16, 24))
    x0, x1 = x0 + ks2, x1 + jnp.int32(2)
    x0, x1 = rounds(x0, x1, (13, 15, 26, 6))
    x0, x1 = x0, x1 + ks1 + jnp.int32(3)
    x0, x1 = rounds(x0, x1, (17, 29, 16, 24))
    x0, x1 = x0 + ks1, x1 + ks2 + jnp.int32(4)
    x0, x1 = rounds(x0, x1, (13, 15, 26, 6))
    x0, x1 = x0 + ks2, x1 + jnp.int32(5)
    bits = x0 ^ x1

    fb = jax.lax.shift_right_logical(bits, jnp.int32(9)) | jnp.int32(0x3F800000)
    floats = jax.lax.bitcast_convert_type(fb, jnp.float32) - jnp.float32(1.0)
    tiny = jnp.float32(np.finfo(np.float32).tiny)
    u = jax.lax.max(tiny, floats + tiny)
    return -jnp.log(-jnp.log(u))


def _sample_body(c_ref, mm_ref, xi_ref, xa_ref, g_ref, idx_ref, logits_s):
    i = pl.program_id(0)
    xi = xi_ref[...]
    BR = xi.shape[0]
    n = xa_ref.shape[0]

    @pl.when(pl.program_id(1) == 0)
    def _():
        log_w, keep = _logw_block(xi, xa_ref[...], i * BR)
        w = jnp.where(keep, jnp.exp(log_w - mm_ref[0, 0]), 0.0)
        wn = w / c_ref[...]
        logits_s[...] = jnp.log(wn + 1e-12)

    vals = logits_s[...] + g_ref[0]
    mx = jnp.max(vals, axis=1, keepdims=True)
    cols = jax.lax.broadcasted_iota(jnp.int32, vals.shape, 1)
    idx = jnp.min(jnp.where(vals == mx, cols, n), axis=1, keepdims=True)
    idx_ref[0] = idx


def _sample_negative_indices(xs):
    n, d = xs.shape
    BR1 = _row_block(n, 256)
    m, s, raw = pl.pallas_call(
        _stats_body,
        grid=(n // BR1,),
        in_specs=[
            pl.BlockSpec((BR1, d), lambda i: (i, 0)),
            pl.BlockSpec((n, d), lambda i: (0, 0)),
        ],
        out_specs=[
            pl.BlockSpec((BR1, 128), lambda i: (i, 0)),
            pl.BlockSpec((BR1, 128), lambda i: (i, 0)),
            pl.BlockSpec((BR1, 128), lambda i: (i, 0)),
        ],
        out_shape=[jax.ShapeDtypeStruct((n, 128), jnp.float32)] * 3,
    )(xs, xs)
    mm = jnp.max(raw[:, 0]).reshape(1, 1)
    c = s[:, :1] * jnp.exp(m[:, :1] - mm) + 1e-8

    g = jax.random.gumbel(jax.random.key(42), (_K - 1, n, n), jnp.float32)

    BR2 = _row_block(n, 128)
    idx = pl.pallas_call(
        _sample_body,
        grid=(n // BR2, _K - 1),
        in_specs=[
            pl.BlockSpec((BR2, 1), lambda i, r: (i, 0)),
            pl.BlockSpec((1, 1), lambda i, r: (0, 0)),
            pl.BlockSpec((BR2, d), lambda i, r: (i, 0)),
            pl.BlockSpec((n, d), lambda i, r: (0, 0)),
            pl.BlockSpec((1, BR2, n), lambda i, r: (r, i, 0)),
        ],
        out_specs=pl.BlockSpec((1, BR2, 1), lambda i, r: (r, i, 0)),
        out_shape=jax.ShapeDtypeStruct((_K - 1, n, 1), jnp.int32),
        scratch_shapes=[pltpu.VMEM((BR2, n), jnp.float32)],
    )(c, mm, xs, xs, g)
    return idx[:, :, 0].T.reshape(-1)


def _sc_sqrt(z):
    bz = plsc.bitcast(z, jnp.int32)
    y = plsc.bitcast(jnp.int32(0x5F3759DF) - jax.lax.shift_right_logical(bz, jnp.int32(1)),
                     jnp.float32)
    half = jnp.float32(0.5) * z
    for _ in range(3):
        y = y * (jnp.float32(1.5) - half * y * y)
    return z * y


def _make_sc_loss(n, d, margin):
    info = plsc.get_sparse_core_info()
    nw = info.num_cores * info.num_subcores
    apw = n // nw
    tpw = apw * (_K - 1)
    chunks = -(-tpw // 128)
    tpad = chunks * 128
    iters = tpad // 16
    slab_rows = min(n, ((apw + 7) // 8) * 8 + 8)
    mesh = plsc.VectorSubcoreMesh(core_axis_name="c", subcore_axis_name="s")

    @functools.partial(
        pl.kernel,
        mesh=mesh,
        compiler_params=pltpu.CompilerParams(needs_layout_passes=False),
        out_type=[
            jax.ShapeDtypeStruct((nw * 16,), jnp.float32),
            jax.ShapeDtypeStruct((nw * 16,), jnp.int32),
        ],
        scratch_types=[
            pltpu.VMEM((slab_rows, d), jnp.float32),
            pltpu.VMEM((tpad, d), jnp.float32),
            pltpu.VMEM((chunks, 128), jnp.int32),
            pltpu.VMEM((slab_rows,), jnp.float32),
            pltpu.VMEM((16,), jnp.float32),
            pltpu.VMEM((16,), jnp.int32),
            pltpu.SemaphoreType.DMA,
        ],
    )
    def sc_loss(x_hbm, nidx_hbm, beta_hbm, sums_hbm, cnts_hbm,
                slab_v, negs_v, idx_v, beta_v, sout_v, cout_v, sem):
        wid = jax.lax.axis_index("s") * info.num_cores + jax.lax.axis_index("c")
        base_a = wid * apw
        slab0 = pl.multiple_of(
            jnp.minimum((base_a // 8) * 8, n - slab_rows), 8)
        off = base_a - slab0
        pltpu.sync_copy(nidx_hbm.at[wid], idx_v)
        gathers = [
            pltpu.async_copy(x_hbm.at[idx_v.at[cchunk]],
                             negs_v.at[pl.ds(cchunk * 128, 128)], sem)
            for cchunk in range(chunks)
        ]
        pltpu.sync_copy(x_hbm.at[pl.ds(slab0, slab_rows)], slab_v)
        pltpu.sync_copy(beta_hbm.at[pl.ds(slab0, slab_rows)], beta_v)
        for cp in gathers:
            cp.wait()

        lane = jax.lax.iota(jnp.int32, 16)

        def tri_body(it, carry):
            s_acc, c_acc = carry
            t16 = it * 16 + lane
            a_loc = jax.lax.shift_right_logical(t16, jnp.int32(2))
            rr = jax.lax.bitwise_and(t16, jnp.int32(3))
            blk = jax.lax.shift_right_logical(a_loc * jnp.int32(52429), jnp.int32(18))
            pos_in = a_loc - blk * jnp.int32(_K)
            mth = jnp.where(rr >= pos_in, rr + jnp.int32(1), rr)
            p_loc = blk * jnp.int32(_K) + mth
            beta16 = plsc.load_gather(beta_v, [off + a_loc])

            a_row = off + a_loc
            p_row = off + p_loc

            def dim_body(dc, dcarry):
                ap, an = dcarry
                for u in range(8):
                    dsplat = jnp.full((16,), dc * 8 + u, jnp.int32)
                    a_d = plsc.load_gather(slab_v, [a_row, dsplat])
                    p_d = plsc.load_gather(slab_v, [p_row, dsplat])
                    n_d = plsc.load_gather(negs_v, [t16, dsplat])
                    ap = ap + (a_d - p_d) * (a_d - p_d)
                    an = an + (a_d - n_d) * (a_d - n_d)
                return (ap, an)

            zero = jnp.zeros((16,), jnp.float32)
            ap, an = plsc.parallel_loop(0, d // 8, carry=(zero, zero))(dim_body)
            d_ap = _sc_sqrt(ap + jnp.float32(1e-8))
            d_an = _sc_sqrt(an + jnp.float32(1e-8))
            pos = jnp.maximum(d_ap - beta16 + jnp.float32(margin), jnp.float32(0.0))
            neg = jnp.maximum(beta16 - d_an + jnp.float32(margin), jnp.float32(0.0))
            valid = t16 < jnp.int32(tpw)
            hit = jnp.logical_and(valid,
                                  jnp.logical_or(pos > jnp.float32(0.0),
                                                 neg > jnp.float32(0.0)))
            s_acc = s_acc + jnp.where(valid, pos + neg, jnp.float32(0.0))
            c_acc = c_acc + jnp.where(hit, jnp.int32(1), jnp.int32(0))
            return (s_acc, c_acc)

        zf = jnp.zeros((16,), jnp.float32)
        zi = jnp.zeros((16,), jnp.int32)
        s_acc, c_acc = plsc.parallel_loop(0, iters, carry=(zf, zi))(tri_body)
        sout_v[...] = s_acc
        cout_v[...] = c_acc
        pltpu.sync_copy(sout_v, sums_hbm.at[pl.ds(wid * 16, 16)])
        pltpu.sync_copy(cout_v, cnts_hbm.at[pl.ds(wid * 16, 16)])

    return sc_loss


def _triplet_indices(n, k):
    a_idx = np.repeat(np.arange(n), k - 1)
    blocks = np.arange(n) // k
    offs = np.arange(k)
    p_full = blocks[:, None] * k + offs[None, :]
    p_keep = p_full != np.arange(n)[:, None]
    p_idx = p_full[p_keep]
    return a_idx, p_idx


def kernel(x, y, beta_in):
    n, d = x.shape
    xs = jax.lax.stop_gradient(x)
    n_index = _sample_negative_indices(xs)

    info = plsc.get_sparse_core_info()
    nw = info.num_cores * info.num_subcores
    tpw = (n // nw) * (_K - 1)
    chunks = -(-tpw // 128)
    nidx = jnp.pad(n_index.reshape(nw, tpw), ((0, 0), (0, chunks * 128 - tpw))
                   ).reshape(nw, chunks, 128).astype(jnp.int32)
    sums, cnts = _make_sc_loss(n, d, _MARGIN)(x, nidx, beta_in)
    pair_cnt = jnp.sum(cnts)
    return jnp.sum(sums) / pair_cnt.astype(jnp.float32)
